# baseline (speedup 1.0000x reference)
import functools

import jax
import jax.numpy as jnp
from jax import lax
from jax.experimental import pallas as pl
from jax.experimental.pallas import tpu as pltpu
from jax.experimental.pallas import tpu_sc as plsc

N_ITEMS = 1000
N_PERSONS = 100000
BATCH = 16384
_NC = 2
_NS = 16
_NW = _NC * _NS
_BPW = BATCH // _NW
_HALF_LOG_2PI = 0.9189385332046727
_N_PARAMS = N_ITEMS + 4 * N_ITEMS + N_PERSONS


def _sp(x):
    return jnp.maximum(x, 0.0) + jnp.log(1.0 + jnp.exp(-jnp.abs(x)))


def _sig(x):
    return 1.0 / (1.0 + jnp.exp(-x))


def _sc_gather(a1, bb1, d01, d11, d21, t, item1, person1):
    mesh = plsc.VectorSubcoreMesh(core_axis_name="c", subcore_axis_name="s")
    out_types = [jax.ShapeDtypeStruct((BATCH,), jnp.float32)
                 for _ in range(6)]
    scratch = (
        [pltpu.VMEM((5120,), jnp.float32)]
        + [pltpu.VMEM((_BPW,), jnp.int32) for _ in range(2)]
        + [pltpu.VMEM((_BPW,), jnp.float32) for _ in range(6)]
        + [pltpu.SemaphoreType.DMA]
    )

    @functools.partial(
        pl.kernel, mesh=mesh, out_type=out_types, scratch_types=scratch,
        compiler_params=pltpu.CompilerParams(
            use_tc_tiling_on_sc=False, needs_layout_passes=False))
    def k(a_h, bb_h, d0_h, d1_h, d2_h, t_h, item_h, person_h,
          oa, obb, od0, od1, od2, ot,
          pk, ii, ip, ba, bbb, bd0, bd1, bd2, bt, sem):
        wid = lax.axis_index("s") * _NC + lax.axis_index("c")
        base = wid * _BPW
        pltpu.sync_copy(person_h.at[pl.ds(base, _BPW)], ip)
        pltpu.sync_copy(item_h.at[pl.ds(base, _BPW)], ii)
        copies = [
            pltpu.async_copy(t_h.at[ip.at[pl.ds(j * 128, 128)]],
                             bt.at[pl.ds(j * 128, 128)], sem)
            for j in range(_BPW // 128)
        ]
        pltpu.sync_copy(a_h, pk.at[pl.ds(0, N_ITEMS)])
        pltpu.sync_copy(bb_h, pk.at[pl.ds(1024, N_ITEMS)])
        pltpu.sync_copy(d0_h, pk.at[pl.ds(2048, N_ITEMS)])
        pltpu.sync_copy(d1_h, pk.at[pl.ds(3072, N_ITEMS)])
        pltpu.sync_copy(d2_h, pk.at[pl.ds(4096, N_ITEMS)])

        def body(i, _):
            sl = pl.ds(i * 16, 16)
            it = ii[sl]
            ba[sl] = plsc.load_gather(pk, [it])
            bbb[sl] = plsc.load_gather(pk, [it + 1024])
            bd0[sl] = plsc.load_gather(pk, [it + 2048])
            bd1[sl] = plsc.load_gather(pk, [it + 3072])
            bd2[sl] = plsc.load_gather(pk, [it + 4096])
            return 0

        lax.fori_loop(0, _BPW // 16, body, 0)
        for c in copies:
            c.wait()
        pltpu.sync_copy(ba, oa.at[pl.ds(base, _BPW)])
        pltpu.sync_copy(bbb, obb.at[pl.ds(base, _BPW)])
        pltpu.sync_copy(bd0, od0.at[pl.ds(base, _BPW)])
        pltpu.sync_copy(bd1, od1.at[pl.ds(base, _BPW)])
        pltpu.sync_copy(bd2, od2.at[pl.ds(base, _BPW)])
        pltpu.sync_copy(bt, ot.at[pl.ds(base, _BPW)])

    return k(a1, bb1, d01, d11, d21, t, item1, person1)


def _final_body(a_ref, bb_ref, d0_ref, d1_ref, d2_ref, t_ref,
                ga_ref, gbb_ref, gd0_ref, gd1_ref, gd2_ref, gt_ref, resp_ref,
                out_ref):
    a = _sp(a_ref[...])
    b0 = bb_ref[...]
    b1 = b0 + _sp(d0_ref[...])
    b2 = b1 + _sp(d1_ref[...])
    b3 = b2 + _sp(d2_ref[...])
    sq = (jnp.sum(a * a) + jnp.sum(b0 * b0 + b1 * b1 + b2 * b2 + b3 * b3)
          + jnp.sum(t_ref[...] ** 2))
    log_prior = -0.5 * sq - _HALF_LOG_2PI * _N_PARAMS

    ai = _sp(ga_ref[...])
    gb0 = gbb_ref[...]
    gb1 = gb0 + _sp(gd0_ref[...])
    gb2 = gb1 + _sp(gd1_ref[...])
    gb3 = gb2 + _sp(gd2_ref[...])
    gt = gt_ref[...]
    r = resp_ref[...]
    bu = jnp.where(r == 2, gb0, jnp.where(r == 3, gb1,
                   jnp.where(r == 4, gb2, gb3)))
    bl = jnp.where(r == 1, gb0, jnp.where(r == 2, gb1,
                   jnp.where(r == 3, gb2, gb3)))
    upper = jnp.where(r == 1, 1.0, _sig(ai * (gt - bu)))
    lower = jnp.where(r == 5, 0.0, _sig(ai * (gt - bl)))
    ll = jnp.sum(jnp.log(upper - lower + 1e-10))

    out_ref[0, 0] = -(ll + log_prior * (BATCH / 1e6))


def kernel(a_, b_base_, b_diff_, t, indices):
    item1 = indices[:, 0]
    person1 = indices[:, 1]
    resp1 = indices[:, 2]
    bb1 = b_base_[:, 0]
    d01 = b_diff_[:, 0]
    d11 = b_diff_[:, 1]
    d21 = b_diff_[:, 2]

    ga, gbb, gd0, gd1, gd2, gt = _sc_gather(
        a_, bb1, d01, d11, d21, t, item1, person1)

    sq128 = lambda x: x.reshape(128, 128)
    pad_neg = lambda x: jnp.pad(x, (0, 24), constant_values=-100.0).reshape(8, 128)
    pad_bb = jnp.pad(bb1, (0, 24)).reshape(8, 128)
    pad_t = jnp.pad(t, (0, 352)).reshape(784, 128)
    out = pl.pallas_call(
        _final_body,
        out_shape=jax.ShapeDtypeStruct((1, 1), jnp.float32),
        out_specs=pl.BlockSpec(memory_space=pltpu.SMEM),
    )(pad_neg(a_), pad_bb, pad_neg(d01), pad_neg(d11), pad_neg(d21), pad_t,
      sq128(ga), sq128(gbb), sq128(gd0), sq128(gd1), sq128(gd2), sq128(gt),
      sq128(resp1))
    return out[0, 0]
